# Initial kernel scaffold; baseline (speedup 1.0000x reference)
import jax
import jax.numpy as jnp
from jax.experimental import pallas as pl


def kernel(x, adj, W0, b0, W1, b1, W2, b2, W3, b3):
    raise NotImplementedError("write your pallas kernel here")


# speedup vs baseline: 8.7523x; 8.7523x over previous
import jax
import jax.numpy as jnp
from jax import lax
from jax.experimental import pallas as pl
from jax.experimental.pallas import tpu as pltpu


def _gcn_body(x_ref, adj_ref, W0, b0, W1, b1, W2, b2, W3, b3, out_ref):
    adj = adj_ref[0]
    N = adj.shape[0]

    rows = lax.broadcasted_iota(jnp.int32, (N, N), 0)
    cols = lax.broadcasted_iota(jnp.int32, (N, N), 1)
    eye = rows == cols
    diag = jnp.sum(jnp.where(eye, adj, 0.0), axis=1, keepdims=True)
    rowsum = jnp.sum(adj, axis=1, keepdims=True)
    deg = jnp.maximum(rowsum - diag + 1.0, 1.0)
    d = lax.rsqrt(deg)
    off = (1.0 - diag) * d

    h = x_ref[0]
    layers = ((W0, b0, True), (W1, b1, True), (W2, b2, True), (W3, b3, False))
    for W_ref, b_ref, act in layers:
        z = jnp.dot(h, W_ref[...], preferred_element_type=jnp.float32)
        zd = z * d
        y = jnp.dot(adj, zd, preferred_element_type=jnp.float32) + off * z
        h = y * d + b_ref[...]
        if act:
            h = jnp.tanh(h)
    out_ref[0] = h


def kernel(x, adj, W0, b0, W1, b1, W2, b2, W3, b3):
    B, N, F_in = x.shape
    F_out = W3.shape[1]
    grid = (B,)
    out = pl.pallas_call(
        _gcn_body,
        grid=grid,
        in_specs=[
            pl.BlockSpec((1, N, F_in), lambda b: (b, 0, 0)),
            pl.BlockSpec((1, N, N), lambda b: (b, 0, 0)),
            pl.BlockSpec(W0.shape, lambda b: (0, 0)),
            pl.BlockSpec((1, W0.shape[1]), lambda b: (0, 0)),
            pl.BlockSpec(W1.shape, lambda b: (0, 0)),
            pl.BlockSpec((1, W1.shape[1]), lambda b: (0, 0)),
            pl.BlockSpec(W2.shape, lambda b: (0, 0)),
            pl.BlockSpec((1, W2.shape[1]), lambda b: (0, 0)),
            pl.BlockSpec(W3.shape, lambda b: (0, 0)),
            pl.BlockSpec((1, W3.shape[1]), lambda b: (0, 0)),
        ],
        out_specs=pl.BlockSpec((1, N, F_out), lambda b: (b, 0, 0)),
        out_shape=jax.ShapeDtypeStruct((B, N, F_out), jnp.float32),
        compiler_params=pltpu.CompilerParams(
            dimension_semantics=("parallel",),
        ),
    )(x, adj, W0, b0.reshape(1, -1), W1, b1.reshape(1, -1),
      W2, b2.reshape(1, -1), W3, b3.reshape(1, -1))
    return out
